# baseline (speedup 1.0000x reference)
import functools

import jax
import jax.numpy as jnp
from jax import lax
from jax.experimental import pallas as pl
from jax.experimental.pallas import tpu as pltpu
from jax.experimental.pallas import tpu_sc as plsc

B, S, D = 4096, 200, 64
L = 16
NC, NS = 2, 16
NW = NC * NS
BBLK = B // NW
POCT = 8
NOCT = S // POCT
PITCH = 137
NG = BBLK // L
VHALF = 500000

_mesh = plsc.VectorSubcoreMesh(
    core_axis_name="c", subcore_axis_name="s", num_cores=NC, num_subcores=NS
)


@functools.partial(
    pl.kernel,
    out_type=jax.ShapeDtypeStruct((S, D, B), jnp.float32),
    mesh=_mesh,
    scratch_types=[
        pltpu.VMEM((POCT, BBLK), jnp.int32),
        pltpu.VMEM((POCT, BBLK), jnp.int32),
        pltpu.VMEM((BBLK, PITCH), jnp.float32),
        pltpu.VMEM((BBLK, PITCH), jnp.float32),
        pltpu.VMEM((D, BBLK), jnp.float32),
        pltpu.VMEM((S, D), jnp.float32),
        pltpu.SemaphoreType.DMA,
        pltpu.SemaphoreType.DMA,
    ],
    compiler_params=pltpu.CompilerParams(
        use_tc_tiling_on_sc=True, needs_layout_passes=False
    ),
)
def _embed(xt_hbm, tok2_hbm, post_hbm, out_hbm, idx_v, idx2_v, rows0_v,
           rows1_v, outb_v, pos_v, gsem, wsem):
    wid = lax.axis_index("s") * NC + lax.axis_index("c")
    b0 = wid * BBLK
    pltpu.sync_copy(post_hbm.at[pl.ds(0, S)], pos_v)
    rows = (rows0_v, rows1_v)
    jvec = lax.iota(jnp.int32, L)

    def stage_octet(o):
        pltpu.sync_copy(xt_hbm.at[pl.ds(o * POCT, POCT), pl.ds(b0, BBLK)], idx_v)
        for pp in range(POCT):
            for g in range(NG):
                raw = idx_v[pp, pl.ds(g * L, L)]
                idx2_v[pp, pl.ds(g * L, L)] = lax.shift_right_logical(raw, 1)

    def fire(pp, buf):
        pltpu.async_copy(tok2_hbm.at[idx2_v.at[pp]], buf.at[:, pl.ds(0, 128)], gsem)

    def drain_gather(buf):
        pltpu.make_async_copy(
            tok2_hbm.at[pl.ds(0, BBLK)], buf.at[:, pl.ds(0, 128)], gsem
        ).wait()

    def drain_write():
        pltpu.make_async_copy(tok2_hbm.at[pl.ds(0, D)], outb_v, wsem).wait()

    def read_pars(pp):
        return tuple(
            lax.mul(
                lax.bitwise_and(idx_v[pp, pl.ds(g * L, L)], jnp.int32(1)),
                jnp.int32(D),
            )
            for g in range(NG)
        )

    def process(p, pars, buf):
        def dg_body(dg, carry):
            d0 = pl.multiple_of(dg * L, L)
            pvec = pos_v[p, pl.ds(d0, L)]
            for k in range(L):
                d = d0 + k
                pv = jnp.broadcast_to(pvec[k], (L,))
                for g in range(NG):
                    vals = plsc.load_gather(
                        buf, [jvec + jnp.int32(g * L), carry[g] + d]
                    )
                    outb_v[d, pl.ds(g * L, L)] = vals + pv
            return carry

        lax.fori_loop(0, D // L, dg_body, pars)
        pltpu.async_copy(outb_v, out_hbm.at[p, :, pl.ds(b0, BBLK)], wsem)

    stage_octet(0)
    fire(0, rows[0])

    def octet_body(o, carry):
        for pp in range(POCT):
            p = o * POCT + pp
            cur = rows[pp % 2]
            nxt = rows[(pp + 1) % 2]
            drain_gather(cur)
            pars = read_pars(pp)

            if pp < POCT - 1:
                fire(pp + 1, nxt)
            else:
                @pl.when(o + 1 < NOCT)
                def _():
                    stage_octet(o + 1)
                    fire(0, nxt)

            @pl.when(p > 0)
            def _():
                drain_write()

            process(p, pars, cur)
        return carry

    lax.fori_loop(0, NOCT, octet_body, 0)
    drain_write()


def kernel(x, token_table, pos_table):
    out_t = _embed(x.T, token_table.reshape(VHALF, 128), pos_table)
    return out_t.transpose(2, 0, 1)


# speedup vs baseline: 1.0166x; 1.0166x over previous
import functools

import jax
import jax.numpy as jnp
from jax import lax
from jax.experimental import pallas as pl
from jax.experimental.pallas import tpu as pltpu
from jax.experimental.pallas import tpu_sc as plsc

B, S, D = 4096, 200, 64
L = 16
NC, NS = 2, 16
NW = NC * NS
BBLK = B // NW
POCT = 8
NOCT = S // POCT
PITCH = 129
NG = BBLK // L
VHALF = 500000

_mesh = plsc.VectorSubcoreMesh(
    core_axis_name="c", subcore_axis_name="s", num_cores=NC, num_subcores=NS
)


@functools.partial(
    pl.kernel,
    out_type=jax.ShapeDtypeStruct((S, D, B), jnp.float32),
    mesh=_mesh,
    scratch_types=[
        pltpu.VMEM((POCT, BBLK), jnp.int32),
        pltpu.VMEM((2, BBLK), jnp.int32),
        pltpu.VMEM((2, BBLK), jnp.int32),
        pltpu.VMEM((BBLK, PITCH), jnp.float32),
        pltpu.VMEM((BBLK, PITCH), jnp.float32),
        pltpu.VMEM((D, BBLK), jnp.float32),
        pltpu.VMEM((D, BBLK), jnp.float32),
        pltpu.VMEM((S, D), jnp.float32),
        pltpu.SemaphoreType.DMA,
        pltpu.SemaphoreType.DMA,
        pltpu.SemaphoreType.DMA,
    ],
    compiler_params=pltpu.CompilerParams(
        use_tc_tiling_on_sc=True, needs_layout_passes=False
    ),
)
def _embed(xt_hbm, tok2_hbm, post_hbm, out_hbm, idx_v, idx2_v, pars_v,
           r0, r1, ob0, ob1, pos_v, gsem, wsem0, wsem1):
    wid = lax.axis_index("s") * NC + lax.axis_index("c")
    b0 = wid * BBLK
    pltpu.sync_copy(post_hbm.at[pl.ds(0, S)], pos_v)
    rows = (r0, r1)
    outbs = (ob0, ob1)
    wsems = (wsem0, wsem1)
    jvs = tuple(lax.iota(jnp.int32, L) + jnp.int32(g * L) for g in range(NG))

    def stage_octet(o):
        pltpu.sync_copy(xt_hbm.at[pl.ds(o * POCT, POCT), pl.ds(b0, BBLK)], idx_v)

    def prep_fire(opp, rpp):
        for g in range(NG):
            raw = idx_v[opp, pl.ds(g * L, L)]
            idx2_v[rpp, pl.ds(g * L, L)] = lax.shift_right_logical(raw, 1)
            pars_v[rpp, pl.ds(g * L, L)] = lax.mul(
                lax.bitwise_and(raw, jnp.int32(1)), jnp.int32(D)
            )
        pltpu.async_copy(
            tok2_hbm.at[idx2_v.at[rpp]], rows[rpp].at[:, pl.ds(0, 128)], gsem
        )

    def drain_gather(rpp):
        pltpu.make_async_copy(
            tok2_hbm.at[pl.ds(0, BBLK)], rows[rpp].at[:, pl.ds(0, 128)], gsem
        ).wait()

    def drain_write(ob, wsem):
        pltpu.make_async_copy(tok2_hbm.at[pl.ds(0, D)], ob, wsem).wait()

    def process(p, rpp, ob, wsem):
        buf = rows[rpp]
        pars = tuple(pars_v[rpp, pl.ds(g * L, L)] for g in range(NG))

        @plsc.parallel_loop(0, D // L, carry=pars)
        def dg_body(dg, carry):
            d0 = pl.multiple_of(dg * L, L)
            pvec = pos_v[p, pl.ds(d0, L)]
            for k in range(L):
                d = d0 + k
                pv = jnp.broadcast_to(pvec[k], (L,))
                for g in range(NG):
                    vals = plsc.load_gather(buf, [jvs[g], carry[g] + d])
                    ob[d, pl.ds(g * L, L)] = vals + pv
            return carry

        pltpu.async_copy(ob, out_hbm.at[p, :, pl.ds(b0, BBLK)], wsem)

    stage_octet(0)
    prep_fire(0, 0)

    def octet_body(o, carry):
        for pp in range(POCT):
            p = o * POCT + pp
            drain_gather(pp % 2)

            if pp == POCT - 1:
                @pl.when(o + 1 < NOCT)
                def _():
                    stage_octet(o + 1)

            @pl.when(p + 1 < S)
            def _():
                prep_fire((pp + 1) % POCT, (pp + 1) % 2)

            @pl.when(p > 1)
            def _():
                drain_write(outbs[pp % 2], wsems[pp % 2])

            process(p, pp % 2, outbs[pp % 2], wsems[pp % 2])
        return carry

    lax.fori_loop(0, NOCT, octet_body, 0)
    drain_write(outbs[0], wsems[0])
    drain_write(outbs[1], wsems[1])


def kernel(x, token_table, pos_table):
    out_t = _embed(x.T, token_table.reshape(VHALF, 128), pos_table)
    return out_t.transpose(2, 0, 1)


# speedup vs baseline: 1.6203x; 1.5939x over previous
import functools

import jax
import jax.numpy as jnp
from jax import lax
from jax.experimental import pallas as pl
from jax.experimental.pallas import tpu as pltpu
from jax.experimental.pallas import tpu_sc as plsc

B, S, D = 4096, 200, 64
L = 16
NC, NS = 2, 16
NW = NC * NS
SEQ_PER_W = B // NW
SEQ_PER_CHUNK = 4
CHUNK = SEQ_PER_CHUNK * S
NCHUNK = SEQ_PER_W // SEQ_PER_CHUNK
SPLITS = ((0, 120), (120, 80))

_mesh = plsc.VectorSubcoreMesh(
    core_axis_name="c", subcore_axis_name="s", num_cores=NC, num_subcores=NS
)


@functools.partial(
    pl.kernel,
    out_type=jax.ShapeDtypeStruct((B, S, D), jnp.float32),
    mesh=_mesh,
    scratch_types=[
        pltpu.VMEM((SEQ_PER_CHUNK, S), jnp.int32),
        pltpu.VMEM((SEQ_PER_CHUNK, S), jnp.int32),
        pltpu.VMEM((SEQ_PER_CHUNK, S, D), jnp.float32),
        pltpu.VMEM((SEQ_PER_CHUNK, S, D), jnp.float32),
        pltpu.VMEM((S, D), jnp.float32),
        pltpu.SemaphoreType.DMA,
        pltpu.SemaphoreType.DMA,
        pltpu.SemaphoreType.DMA,
        pltpu.SemaphoreType.DMA,
    ],
    compiler_params=pltpu.CompilerParams(use_tc_tiling_on_sc=False),
)
def _embed(x_hbm, tok_hbm, pos_hbm, out_hbm, idx0, idx1, rows0, rows1,
           pos_v, gs0, gs1, ws0, ws1):
    wid = lax.axis_index("s") * NC + lax.axis_index("c")
    pltpu.sync_copy(pos_hbm, pos_v)
    idxs = (idx0, idx1)
    rows = (rows0, rows1)
    gsems = (gs0, gs1)
    wsems = (ws0, ws1)

    def fire_gather(c, nb):
        seq0 = wid * SEQ_PER_W + c * SEQ_PER_CHUNK
        pltpu.sync_copy(x_hbm.at[pl.ds(seq0, SEQ_PER_CHUNK)], idxs[nb])
        for s0 in range(SEQ_PER_CHUNK):
            for off, n in SPLITS:
                pltpu.async_copy(
                    tok_hbm.at[idxs[nb].at[s0, pl.ds(off, n)]],
                    rows[nb].at[s0, pl.ds(off, n)],
                    gsems[nb],
                )

    def drain_gather(nb):
        for s0 in range(SEQ_PER_CHUNK):
            for off, n in SPLITS:
                pltpu.make_async_copy(
                    tok_hbm.at[pl.ds(0, n)],
                    rows[nb].at[s0, pl.ds(off, n)],
                    gsems[nb],
                ).wait()

    def drain_write(c, nb):
        seq0 = wid * SEQ_PER_W + c * SEQ_PER_CHUNK
        pltpu.make_async_copy(
            rows[nb], out_hbm.at[pl.ds(seq0, SEQ_PER_CHUNK)], wsems[nb]
        ).wait()

    def add_pos(nb):
        def pos_body(p, carry):
            for j in range(D // L):
                pv = pos_v[p, pl.ds(j * L, L)]
                for s0 in range(SEQ_PER_CHUNK):
                    rows[nb][s0, p, pl.ds(j * L, L)] += pv
            return carry

        lax.fori_loop(0, S, pos_body, 0)

    def fire_write(c, nb):
        seq0 = wid * SEQ_PER_W + c * SEQ_PER_CHUNK
        pltpu.async_copy(
            rows[nb], out_hbm.at[pl.ds(seq0, SEQ_PER_CHUNK)], wsems[nb]
        )

    fire_gather(0, 0)

    def chunk_pair(cc, carry):
        for nb in range(2):
            c = cc * 2 + nb
            drain_gather(nb)

            @pl.when(c + 1 < NCHUNK)
            def _():
                @pl.when(c > 0)
                def _():
                    drain_write(c - 1, 1 - nb)

                fire_gather(c + 1, 1 - nb)

            add_pos(nb)
            fire_write(c, nb)
        return carry

    lax.fori_loop(0, NCHUNK // 2, chunk_pair, 0)
    drain_write(NCHUNK - 2, 0)
    drain_write(NCHUNK - 1, 1)


def kernel(x, token_table, pos_table):
    return _embed(x, token_table, pos_table[:S])


# speedup vs baseline: 1.6260x; 1.0035x over previous
import functools

import jax
import jax.numpy as jnp
from jax import lax
from jax.experimental import pallas as pl
from jax.experimental.pallas import tpu as pltpu
from jax.experimental.pallas import tpu_sc as plsc

B, S, D = 4096, 200, 64
L = 16
NC, NS = 2, 16
NW = NC * NS
SEQ_PER_W = B // NW
SEQ_PER_CHUNK = 4
CHUNK = SEQ_PER_CHUNK * S
NCHUNK = SEQ_PER_W // SEQ_PER_CHUNK
SPLITS = ((0, 120), (120, 80))

_mesh = plsc.VectorSubcoreMesh(
    core_axis_name="c", subcore_axis_name="s", num_cores=NC, num_subcores=NS
)


@functools.partial(
    pl.kernel,
    out_type=jax.ShapeDtypeStruct((B, S, D), jnp.float32),
    mesh=_mesh,
    scratch_types=[
        pltpu.VMEM((SEQ_PER_CHUNK, S), jnp.int32),
        pltpu.VMEM((SEQ_PER_CHUNK, S), jnp.int32),
        pltpu.VMEM((SEQ_PER_CHUNK, S, D), jnp.float32),
        pltpu.VMEM((SEQ_PER_CHUNK, S, D), jnp.float32),
        pltpu.VMEM((S, D), jnp.float32),
        pltpu.SemaphoreType.DMA,
        pltpu.SemaphoreType.DMA,
        pltpu.SemaphoreType.DMA,
        pltpu.SemaphoreType.DMA,
    ],
    compiler_params=pltpu.CompilerParams(use_tc_tiling_on_sc=False),
)
def _embed(x_hbm, tok_hbm, pos_hbm, out_hbm, idx0, idx1, rows0, rows1,
           pos_v, gs0, gs1, ws0, ws1):
    wid = lax.axis_index("s") * NC + lax.axis_index("c")
    pltpu.sync_copy(pos_hbm, pos_v)
    idxs = (idx0, idx1)
    rows = (rows0, rows1)
    gsems = (gs0, gs1)
    wsems = (ws0, ws1)

    def fire_gather(c, nb):
        seq0 = wid * SEQ_PER_W + c * SEQ_PER_CHUNK
        pltpu.sync_copy(x_hbm.at[pl.ds(seq0, SEQ_PER_CHUNK)], idxs[nb])
        for s0 in range(SEQ_PER_CHUNK):
            for off, n in SPLITS:
                pltpu.async_copy(
                    tok_hbm.at[idxs[nb].at[s0, pl.ds(off, n)]],
                    rows[nb].at[s0, pl.ds(off, n)],
                    gsems[nb],
                )

    def drain_gather(nb):
        for s0 in range(SEQ_PER_CHUNK):
            for off, n in SPLITS:
                pltpu.make_async_copy(
                    tok_hbm.at[pl.ds(0, n)],
                    rows[nb].at[s0, pl.ds(off, n)],
                    gsems[nb],
                ).wait()

    def drain_write(c, nb):
        seq0 = wid * SEQ_PER_W + c * SEQ_PER_CHUNK
        pltpu.make_async_copy(
            rows[nb], out_hbm.at[pl.ds(seq0, SEQ_PER_CHUNK)], wsems[nb]
        ).wait()

    def add_pos(nb):
        @plsc.parallel_loop(0, S)
        def pos_body(p):
            for j in range(D // L):
                pv = pos_v[p, pl.ds(j * L, L)]
                for s0 in range(SEQ_PER_CHUNK):
                    rows[nb][s0, p, pl.ds(j * L, L)] += pv

    def fire_write(c, nb):
        seq0 = wid * SEQ_PER_W + c * SEQ_PER_CHUNK
        pltpu.async_copy(
            rows[nb], out_hbm.at[pl.ds(seq0, SEQ_PER_CHUNK)], wsems[nb]
        )

    fire_gather(0, 0)

    def chunk_pair(cc, carry):
        for nb in range(2):
            c = cc * 2 + nb
            drain_gather(nb)

            @pl.when(c + 1 < NCHUNK)
            def _():
                @pl.when(c > 0)
                def _():
                    drain_write(c - 1, 1 - nb)

                fire_gather(c + 1, 1 - nb)

            add_pos(nb)
            fire_write(c, nb)
        return carry

    lax.fori_loop(0, NCHUNK // 2, chunk_pair, 0)
    drain_write(NCHUNK - 2, 0)
    drain_write(NCHUNK - 1, 1)


def kernel(x, token_table, pos_table):
    return _embed(x, token_table, pos_table[:S])
